# baseline (speedup 1.0000x reference)
import functools

import jax
import jax.numpy as jnp
from jax import lax
from jax.experimental import pallas as pl
from jax.experimental.pallas import tpu as pltpu
from jax.experimental.pallas import tpu_sc as plsc

N_NODES = 50000
D_HID = 16

NC = 2
NS = 16
CHUNK = 128
IB = 16
NB = 49
NW = NC * NS
NR = NW * NB * IB
PE = NR * CHUNK
NRB = 12
GPD = 10
N_ACC = 50048
NDUMP = N_ACC - N_NODES
ZROWS = N_ACC // NS


def _mm2_body(x_ref, w_ref, o1_ref, o2_ref):
    z = jnp.dot(x_ref[...], w_ref[...], preferred_element_type=jnp.float32)
    o1_ref[...] = z[:, :D_HID]
    o2_ref[...] = z[:, D_HID:]


def _mm2(x, w):
    m, k = x.shape
    bm = 2000
    return pl.pallas_call(
        _mm2_body,
        grid=(m // bm,),
        in_specs=[
            pl.BlockSpec((bm, k), lambda i: (i, 0)),
            pl.BlockSpec((k, 2 * D_HID), lambda i: (0, 0)),
        ],
        out_specs=[
            pl.BlockSpec((bm, D_HID), lambda i: (i, 0)),
            pl.BlockSpec((bm, D_HID), lambda i: (i, 0)),
        ],
        out_shape=[
            jax.ShapeDtypeStruct((m, D_HID), jnp.float32),
            jax.ShapeDtypeStruct((m, D_HID), jnp.float32),
        ],
    )(x, w)


def _h1_body(r_ref, p0_ref, p1_ref, b_ref, o_ref):
    v = r_ref[...] + p0_ref[0] + p1_ref[0] + b_ref[...]
    o_ref[...] = jnp.where(v > 0, v, jnp.exp(jnp.minimum(v, 0.0)) - 1.0)


def _h1(r, p, b):
    m = r.shape[0]
    bm = 5000
    return pl.pallas_call(
        _h1_body,
        grid=(m // bm,),
        in_specs=[
            pl.BlockSpec((bm, D_HID), lambda i: (i, 0)),
            pl.BlockSpec((1, bm, D_HID), lambda i: (0, i, 0)),
            pl.BlockSpec((1, bm, D_HID), lambda i: (1, i, 0)),
            pl.BlockSpec((1, D_HID), lambda i: (0, 0)),
        ],
        out_specs=pl.BlockSpec((bm, D_HID), lambda i: (i, 0)),
        out_shape=jax.ShapeDtypeStruct((m, D_HID), jnp.float32),
    )(r, p, p, b)


def _out_body(h_ref, p0_ref, p1_ref, wr_ref, wn_ref, b_ref, o_ref):
    h = h_ref[...]
    agg = p0_ref[0] + p1_ref[0]
    z = (jnp.dot(h, wr_ref[...], preferred_element_type=jnp.float32)
         + jnp.dot(agg, wn_ref[...], preferred_element_type=jnp.float32)
         + b_ref[...])
    mx = jnp.max(z, axis=1, keepdims=True)
    lse = jnp.log(jnp.sum(jnp.exp(z - mx), axis=1, keepdims=True)) + mx
    o_ref[...] = z - lse


def _out_layer(h, p, wr, wn, b):
    m = h.shape[0]
    d_out = wr.shape[1]
    bm = 5000
    return pl.pallas_call(
        _out_body,
        grid=(m // bm,),
        in_specs=[
            pl.BlockSpec((bm, D_HID), lambda i: (i, 0)),
            pl.BlockSpec((1, bm, D_HID), lambda i: (0, i, 0)),
            pl.BlockSpec((1, bm, D_HID), lambda i: (1, i, 0)),
            pl.BlockSpec((D_HID, d_out), lambda i: (0, 0)),
            pl.BlockSpec((D_HID, d_out), lambda i: (0, 0)),
            pl.BlockSpec((1, d_out), lambda i: (0, 0)),
        ],
        out_specs=pl.BlockSpec((bm, d_out), lambda i: (i, 0)),
        out_shape=jax.ShapeDtypeStruct((m, d_out), jnp.float32),
    )(h, p, p, wr, wn, b)


def _segsum(table, ei3):
    mesh = plsc.VectorSubcoreMesh(core_axis_name="c", subcore_axis_name="s")

    @functools.partial(
        pl.kernel,
        out_type=jax.ShapeDtypeStruct((NC, N_ACC, D_HID), jnp.float32),
        mesh=mesh,
        scratch_types=[
            [pltpu.VMEM((IB, CHUNK), jnp.int32) for _ in range(2)],
            [pltpu.VMEM((IB, CHUNK), jnp.int32) for _ in range(2)],
            [pltpu.VMEM((CHUNK, D_HID), jnp.float32) for _ in range(NRB)],
            pltpu.VMEM((ZROWS // 8, D_HID), jnp.float32),
            pltpu.VMEM_SHARED((N_ACC, D_HID), jnp.float32),
            [pltpu.SemaphoreType.DMA for _ in range(2)],
            [pltpu.SemaphoreType.DMA for _ in range(NRB)],
            [pltpu.SemaphoreType.DMA for _ in range(NRB)],
        ],
        compiler_params=pltpu.CompilerParams(use_tc_tiling_on_sc=False),
    )
    def k(table_hbm, ei_hbm, out_hbm,
          sidxs, didxs, rbs, zbuf, acc, isems, gsems, ssems):
        cc = lax.axis_index("c")
        ss = lax.axis_index("s")
        wid = cc * NS + ss

        zero16 = jnp.zeros((D_HID,), jnp.float32)
        zr = ZROWS // 8

        def zfill(i, carry):
            zbuf[i] = zero16
            return carry

        lax.fori_loop(0, zr, zfill, 0)
        for u in range(8):
            pltpu.sync_copy(zbuf, acc.at[pl.ds(ss * ZROWS + u * zr, zr)])
        plsc.subcore_barrier()

        def idx_start(b, p):
            r0 = (wid * NB + b) * IB
            pltpu.async_copy(ei_hbm.at[0, pl.ds(r0, IB)], sidxs[p], isems[p])
            pltpu.async_copy(ei_hbm.at[1, pl.ds(r0, IB)], didxs[p], isems[p])

        def idx_wait(b, p):
            r0 = (wid * NB + b) * IB
            pltpu.make_async_copy(
                ei_hbm.at[0, pl.ds(r0, IB)], sidxs[p], isems[p]).wait()
            pltpu.make_async_copy(
                ei_hbm.at[1, pl.ds(r0, IB)], didxs[p], isems[p]).wait()

        def process(p):
            sidx, didx = sidxs[p], didxs[p]
            gd = [None] * NRB
            sd = [None] * NRB

            def gstart(j):
                q = j % NRB
                if sd[q] is not None:
                    sd[q].wait()
                    sd[q] = None
                gd[q] = pltpu.async_copy(
                    table_hbm.at[sidx.at[j]], rbs[q], gsems[q])

            for j in range(GPD):
                gstart(j)
            for j in range(IB):
                q = j % NRB
                gd[q].wait()
                sd[q] = pltpu.async_copy(
                    rbs[q], acc.at[didx.at[j]], ssems[q], add=True)
                if j + GPD < IB:
                    gstart(j + GPD)
            for q in range(NRB):
                if sd[q] is not None:
                    sd[q].wait()

        idx_start(0, 0)

        def pair(kk, carry):
            b0 = 2 * kk
            idx_wait(b0, 0)
            idx_start(b0 + 1, 1)
            process(0)
            idx_wait(b0 + 1, 1)
            idx_start(b0 + 2, 0)
            process(1)
            return carry

        lax.fori_loop(0, (NB - 1) // 2, pair, 0)
        idx_wait(NB - 1, 0)
        process(0)
        plsc.subcore_barrier()
        pltpu.sync_copy(acc.at[pl.ds(ss * ZROWS, ZROWS)],
                        out_hbm.at[cc].at[pl.ds(ss * ZROWS, ZROWS)])

    return k(table, ei3)


def kernel(x, edge_index, W1_root, W1_nbr, b1, W2_root, W2_nbr, b2):
    e = edge_index.shape[1]
    pad = PE - e
    padv = jnp.arange(pad, dtype=jnp.int32)
    src = jnp.concatenate([edge_index[0], padv % jnp.int32(N_NODES)])
    dst = jnp.concatenate(
        [edge_index[1], jnp.int32(N_NODES) + padv % jnp.int32(NDUMP)])
    ei3 = jnp.stack([src, dst]).reshape(2, NR, CHUNK)

    w1 = jnp.concatenate([W1_nbr, W1_root], axis=1)
    m1, r1 = _mm2(x, w1)

    p1 = _segsum(m1, ei3)
    h1 = _h1(r1, p1, b1.reshape(1, D_HID))

    ph = _segsum(h1, ei3)
    return _out_layer(h1, ph, W2_root, W2_nbr, b2.reshape(1, -1))


# speedup vs baseline: 1.0241x; 1.0241x over previous
import functools

import jax
import jax.numpy as jnp
from jax import lax
from jax.experimental import pallas as pl
from jax.experimental.pallas import tpu as pltpu
from jax.experimental.pallas import tpu_sc as plsc

N_NODES = 50000
D_HID = 16

NC = 2
NS = 16
CHUNK = 128
IB = 24
NB = 33
NW = NC * NS
NR = NW * NB * IB
PE = NR * CHUNK
NRB = 12
GPD = 10
N_ACC = 50048
NDUMP = N_ACC - N_NODES
ZROWS = N_ACC // NS


def _mm2_body(x_ref, w_ref, o1_ref, o2_ref):
    z = jnp.dot(x_ref[...], w_ref[...], preferred_element_type=jnp.float32)
    o1_ref[...] = z[:, :D_HID]
    o2_ref[...] = z[:, D_HID:]


def _mm2(x, w):
    m, k = x.shape
    bm = 2000
    return pl.pallas_call(
        _mm2_body,
        grid=(m // bm,),
        in_specs=[
            pl.BlockSpec((bm, k), lambda i: (i, 0)),
            pl.BlockSpec((k, 2 * D_HID), lambda i: (0, 0)),
        ],
        out_specs=[
            pl.BlockSpec((bm, D_HID), lambda i: (i, 0)),
            pl.BlockSpec((bm, D_HID), lambda i: (i, 0)),
        ],
        out_shape=[
            jax.ShapeDtypeStruct((m, D_HID), jnp.float32),
            jax.ShapeDtypeStruct((m, D_HID), jnp.float32),
        ],
    )(x, w)


def _h1_body(r_ref, p0_ref, p1_ref, b_ref, o_ref):
    v = r_ref[...] + p0_ref[0] + p1_ref[0] + b_ref[...]
    o_ref[...] = jnp.where(v > 0, v, jnp.exp(jnp.minimum(v, 0.0)) - 1.0)


def _h1(r, p, b):
    m = r.shape[0]
    bm = 5000
    return pl.pallas_call(
        _h1_body,
        grid=(m // bm,),
        in_specs=[
            pl.BlockSpec((bm, D_HID), lambda i: (i, 0)),
            pl.BlockSpec((1, bm, D_HID), lambda i: (0, i, 0)),
            pl.BlockSpec((1, bm, D_HID), lambda i: (1, i, 0)),
            pl.BlockSpec((1, D_HID), lambda i: (0, 0)),
        ],
        out_specs=pl.BlockSpec((bm, D_HID), lambda i: (i, 0)),
        out_shape=jax.ShapeDtypeStruct((m, D_HID), jnp.float32),
    )(r, p, p, b)


def _out_body(h_ref, p0_ref, p1_ref, wr_ref, wn_ref, b_ref, o_ref):
    h = h_ref[...]
    agg = p0_ref[0] + p1_ref[0]
    z = (jnp.dot(h, wr_ref[...], preferred_element_type=jnp.float32)
         + jnp.dot(agg, wn_ref[...], preferred_element_type=jnp.float32)
         + b_ref[...])
    mx = jnp.max(z, axis=1, keepdims=True)
    lse = jnp.log(jnp.sum(jnp.exp(z - mx), axis=1, keepdims=True)) + mx
    o_ref[...] = z - lse


def _out_layer(h, p, wr, wn, b):
    m = h.shape[0]
    d_out = wr.shape[1]
    bm = 5000
    return pl.pallas_call(
        _out_body,
        grid=(m // bm,),
        in_specs=[
            pl.BlockSpec((bm, D_HID), lambda i: (i, 0)),
            pl.BlockSpec((1, bm, D_HID), lambda i: (0, i, 0)),
            pl.BlockSpec((1, bm, D_HID), lambda i: (1, i, 0)),
            pl.BlockSpec((D_HID, d_out), lambda i: (0, 0)),
            pl.BlockSpec((D_HID, d_out), lambda i: (0, 0)),
            pl.BlockSpec((1, d_out), lambda i: (0, 0)),
        ],
        out_specs=pl.BlockSpec((bm, d_out), lambda i: (i, 0)),
        out_shape=jax.ShapeDtypeStruct((m, d_out), jnp.float32),
    )(h, p, p, wr, wn, b)


def _segsum(table, ei3):
    mesh = plsc.VectorSubcoreMesh(core_axis_name="c", subcore_axis_name="s")

    @functools.partial(
        pl.kernel,
        out_type=jax.ShapeDtypeStruct((NC, N_ACC, D_HID), jnp.float32),
        mesh=mesh,
        scratch_types=[
            [pltpu.VMEM((IB, CHUNK), jnp.int32) for _ in range(2)],
            [pltpu.VMEM((IB, CHUNK), jnp.int32) for _ in range(2)],
            [pltpu.VMEM((CHUNK, D_HID), jnp.float32) for _ in range(NRB)],
            pltpu.VMEM((ZROWS // 8, D_HID), jnp.float32),
            pltpu.VMEM_SHARED((N_ACC, D_HID), jnp.float32),
            [pltpu.SemaphoreType.DMA for _ in range(2)],
            [pltpu.SemaphoreType.DMA for _ in range(NRB)],
            [pltpu.SemaphoreType.DMA for _ in range(NRB)],
        ],
        compiler_params=pltpu.CompilerParams(use_tc_tiling_on_sc=False),
    )
    def k(table_hbm, ei_hbm, out_hbm,
          sidxs, didxs, rbs, zbuf, acc, isems, gsems, ssems):
        cc = lax.axis_index("c")
        ss = lax.axis_index("s")
        wid = cc * NS + ss

        zero16 = jnp.zeros((D_HID,), jnp.float32)
        zr = ZROWS // 8

        def zfill(i, carry):
            zbuf[i] = zero16
            return carry

        lax.fori_loop(0, zr, zfill, 0)
        for u in range(8):
            pltpu.sync_copy(zbuf, acc.at[pl.ds(ss * ZROWS + u * zr, zr)])
        plsc.subcore_barrier()

        def idx_start(b, p):
            r0 = (wid * NB + b) * IB
            pltpu.async_copy(ei_hbm.at[0, pl.ds(r0, IB)], sidxs[p], isems[p])
            pltpu.async_copy(ei_hbm.at[1, pl.ds(r0, IB)], didxs[p], isems[p])

        def idx_wait(b, p):
            r0 = (wid * NB + b) * IB
            pltpu.make_async_copy(
                ei_hbm.at[0, pl.ds(r0, IB)], sidxs[p], isems[p]).wait()
            pltpu.make_async_copy(
                ei_hbm.at[1, pl.ds(r0, IB)], didxs[p], isems[p]).wait()

        def process(p):
            sidx, didx = sidxs[p], didxs[p]
            gd = [None] * NRB
            sd = [None] * NRB

            def gstart(j):
                q = j % NRB
                if sd[q] is not None:
                    sd[q].wait()
                    sd[q] = None
                gd[q] = pltpu.async_copy(
                    table_hbm.at[sidx.at[j]], rbs[q], gsems[q])

            for j in range(GPD):
                gstart(j)
            for j in range(IB):
                q = j % NRB
                gd[q].wait()
                sd[q] = pltpu.async_copy(
                    rbs[q], acc.at[didx.at[j]], ssems[q], add=True)
                if j + GPD < IB:
                    gstart(j + GPD)
            for q in range(NRB):
                if sd[q] is not None:
                    sd[q].wait()

        idx_start(0, 0)

        def pair(kk, carry):
            b0 = 2 * kk
            idx_wait(b0, 0)
            idx_start(b0 + 1, 1)
            process(0)
            idx_wait(b0 + 1, 1)
            idx_start(b0 + 2, 0)
            process(1)
            return carry

        lax.fori_loop(0, (NB - 1) // 2, pair, 0)
        idx_wait(NB - 1, 0)
        process(0)
        plsc.subcore_barrier()
        pltpu.sync_copy(acc.at[pl.ds(ss * ZROWS, ZROWS)],
                        out_hbm.at[cc].at[pl.ds(ss * ZROWS, ZROWS)])

    return k(table, ei3)


def kernel(x, edge_index, W1_root, W1_nbr, b1, W2_root, W2_nbr, b2):
    e = edge_index.shape[1]
    pad = PE - e
    padv = jnp.arange(pad, dtype=jnp.int32)
    src = jnp.concatenate([edge_index[0], padv % jnp.int32(N_NODES)])
    dst = jnp.concatenate(
        [edge_index[1], jnp.int32(N_NODES) + padv % jnp.int32(NDUMP)])
    ei3 = jnp.stack([src, dst]).reshape(2, NR, CHUNK)

    w1 = jnp.concatenate([W1_nbr, W1_root], axis=1)
    m1, r1 = _mm2(x, w1)

    p1 = _segsum(m1, ei3)
    h1 = _h1(r1, p1, b1.reshape(1, D_HID))

    ph = _segsum(h1, ei3)
    return _out_layer(h1, ph, W2_root, W2_nbr, b2.reshape(1, -1))
